# Initial kernel scaffold; baseline (speedup 1.0000x reference)
import jax
import jax.numpy as jnp
from jax.experimental import pallas as pl


def kernel(featureMaps, tatumsBoundaries):
    raise NotImplementedError("write your pallas kernel here")


# speedup vs baseline: 47.2245x; 47.2245x over previous
import functools

import jax
import jax.numpy as jnp
from jax import lax
from jax.experimental import pallas as pl
from jax.experimental.pallas import tpu as pltpu
from jax.experimental.pallas import tpu_sc as plsc

B, F, D, T = 4, 2048, 128, 256
K = 15
NW = 32
TPW = (B * T) // NW
LANES = 16
CPT = D // LANES

_mesh = plsc.VectorSubcoreMesh(core_axis_name="c", subcore_axis_name="s")


@functools.partial(
    pl.kernel,
    mesh=_mesh,
    out_type=jax.ShapeDtypeStruct((B * T, D), jnp.float32),
    scratch_types=[
        pltpu.VMEM((TPW,), jnp.int32),
        pltpu.VMEM((TPW,), jnp.int32),
        pltpu.VMEM((K, TPW), jnp.int32),
        pltpu.VMEM((K, TPW, D), jnp.float32),
        pltpu.VMEM((TPW, D), jnp.float32),
        pltpu.SemaphoreType.DMA,
    ],
)
def _tatum_pool_sc(feat_hbm, starts_hbm, stops_hbm, out_hbm,
                   sv, ev, idxv, rowsv, outv, sem):
    c = lax.axis_index("c")
    s = lax.axis_index("s")
    w = c * 16 + s
    b = w // (T // TPW)
    t0 = (w % (T // TPW)) * TPW

    pltpu.sync_copy(starts_hbm.at[b, pl.ds(t0, TPW)], sv)
    pltpu.sync_copy(stops_hbm.at[b, pl.ds(t0, TPW)], ev)

    row0 = b * F
    for j in range(TPW // LANES):
        svec = sv[pl.ds(j * LANES, LANES)] + row0
        emax = ev[pl.ds(j * LANES, LANES)] + (row0 - 1)
        for k in range(K):
            idxv[k, pl.ds(j * LANES, LANES)] = jnp.minimum(svec + k, emax)

    copies = [
        pltpu.async_copy(feat_hbm.at[idxv.at[k]], rowsv.at[k], sem)
        for k in range(K)
    ]
    for cp in copies:
        cp.wait()

    def body(t, carry):
        for cc in range(CPT):
            acc = rowsv[0, t, pl.ds(cc * LANES, LANES)]
            for k in range(1, K):
                acc = jnp.maximum(acc, rowsv[k, t, pl.ds(cc * LANES, LANES)])
            outv[t, pl.ds(cc * LANES, LANES)] = acc
        return carry

    lax.fori_loop(0, TPW, body, 0)

    pltpu.sync_copy(outv, out_hbm.at[pl.ds(w * TPW, TPW)])


def kernel(featureMaps, tatumsBoundaries):
    feat2d = featureMaps.reshape(B * F, D)
    starts = tatumsBoundaries[..., 0].astype(jnp.int32)
    stops = tatumsBoundaries[..., 1].astype(jnp.int32)
    out = _tatum_pool_sc(feat2d, starts, stops)
    return out.reshape(B, T, D)
